# baseline (speedup 1.0000x reference)
import functools

import jax
import jax.numpy as jnp
from jax import lax
from jax.experimental import pallas as pl
from jax.experimental.pallas import tpu as pltpu
from jax.experimental.pallas import tpu_sc as plsc

_MID_CHUNK = 1024
_RAMP = [128, 128, 256, 512]
_NBUF = 6
_LOOKAHEAD = 4


def _chunk_schedule(batch, seq):
    chunks = []
    for b in range(batch):
        sizes = []
        if b == 0:
            sizes += _RAMP
        tail = sum(_RAMP) if b == batch - 1 else 0
        n_mid = (seq - sum(sizes) - tail) // _MID_CHUNK
        sizes += [_MID_CHUNK] * n_mid
        if b == batch - 1:
            sizes += list(reversed(_RAMP))
        row = 0
        for s in sizes:
            chunks.append((b, row, s))
            row += s
    return chunks


def _stream_kernel(pos_ref, emb_ref, hs_ref, out_hs_ref,
                   buf, sem_in, sem_out, sem_ctx):
    batch, seq, _ = hs_ref.shape
    n_ctx = emb_ref.shape[1]
    sched = _chunk_schedule(batch, seq)
    n = len(sched)

    def fetch(i):
        b, row, sz = sched[i]
        m = i % _NBUF
        return pltpu.make_async_copy(
            hs_ref.at[pl.ds(b, 1), pl.ds(row, sz), :],
            buf.at[pl.ds(m, 1), pl.ds(0, sz), :], sem_in.at[m])

    def write(i):
        b, row, sz = sched[i]
        m = i % _NBUF
        return pltpu.make_async_copy(
            buf.at[pl.ds(m, 1), pl.ds(0, sz), :],
            out_hs_ref.at[pl.ds(b, 1), pl.ds(n_ctx + row, sz), :],
            sem_out.at[m])

    def ctx_copy(b2):
        p = pos_ref[0]
        return pltpu.make_async_copy(
            emb_ref.at[pl.ds(p, 1)],
            out_hs_ref.at[pl.ds(b2, 1), pl.ds(0, n_ctx), :], sem_ctx)

    for i in range(_LOOKAHEAD):
        fetch(i).start()

    for b2 in range(batch):
        ctx_copy(b2).start()

    for i in range(n):
        j = i + _LOOKAHEAD
        if j < n:
            if j - _NBUF >= 0:
                write(j - _NBUF).wait()
            fetch(j).start()
        fetch(i).wait()
        write(i).start()

    for i in range(max(0, n - _NBUF), n):
        write(i).wait()
    for b2 in range(batch):
        ctx_copy(b2).wait()


def _extend_hidden(hidden_states, position, embeddings):
    B, S, H = hidden_states.shape
    D, T, _ = embeddings.shape
    pos = jnp.asarray(position, dtype=jnp.int32).reshape((1,))
    return pl.pallas_call(
        _stream_kernel,
        in_specs=[
            pl.BlockSpec(memory_space=pltpu.SMEM),
            pl.BlockSpec(memory_space=pl.ANY),
            pl.BlockSpec(memory_space=pl.ANY),
        ],
        out_specs=pl.BlockSpec(memory_space=pl.ANY),
        out_shape=jax.ShapeDtypeStruct((B, T + S, H), hidden_states.dtype),
        scratch_shapes=[
            pltpu.VMEM((_NBUF, _MID_CHUNK, H), hidden_states.dtype),
            pltpu.SemaphoreType.DMA((_NBUF,)),
            pltpu.SemaphoreType.DMA((_NBUF,)),
            pltpu.SemaphoreType.DMA,
        ],
    )(pos, embeddings, hidden_states)


def _extend_mask(attention_mask, n_ctx):
    B, S = attention_mask.shape
    dt = attention_mask.dtype
    mesh = plsc.VectorSubcoreMesh(core_axis_name="c", subcore_axis_name="s")

    @functools.partial(
        pl.kernel,
        out_type=jax.ShapeDtypeStruct((B * (n_ctx + S),), dt),
        mesh=mesh,
        scratch_types=[
            pltpu.VMEM((S,), dt),
            pltpu.VMEM((n_ctx,), dt),
        ],
    )
    def mask_kernel(mask_hbm, out_hbm, row_buf, ones_buf):
        c = lax.axis_index("c")
        s = lax.axis_index("s")
        nc = plsc.get_sparse_core_info().num_cores
        wid = s * nc + c

        for b in range(B):
            @pl.when(wid == b)
            def _(b=b):
                ones_buf[...] = jnp.ones((n_ctx,), dt)
                pltpu.sync_copy(mask_hbm.at[pl.ds(b * S, S)], row_buf)
                pltpu.sync_copy(
                    ones_buf, out_hbm.at[pl.ds(b * (n_ctx + S), n_ctx)])
                pltpu.sync_copy(
                    row_buf, out_hbm.at[pl.ds(b * (n_ctx + S) + n_ctx, S)])

    flat = mask_kernel(attention_mask.reshape((B * S,)))
    return flat.reshape((B, n_ctx + S))


def kernel(hidden_states, attention_mask, position, embeddings):
    T = embeddings.shape[1]
    out_hs = _extend_hidden(hidden_states, position, embeddings)
    out_mask = _extend_mask(attention_mask, T)
    return out_hs, out_mask


# speedup vs baseline: 1.1987x; 1.1987x over previous
import jax
import jax.numpy as jnp
from jax.experimental import pallas as pl
from jax.experimental.pallas import tpu as pltpu

_MID_CHUNK = 1024
_RAMP = [128, 128, 256, 512]
_NBUF = 6
_LOOKAHEAD = 4


def _chunk_schedule(batch, seq):
    chunks = []
    for b in range(batch):
        sizes = []
        if b == 0:
            sizes += _RAMP
        tail = sum(_RAMP) if b == batch - 1 else 0
        n_mid = (seq - sum(sizes) - tail) // _MID_CHUNK
        sizes += [_MID_CHUNK] * n_mid
        if b == batch - 1:
            sizes += list(reversed(_RAMP))
        row = 0
        for s in sizes:
            chunks.append((b, row, s))
            row += s
    return chunks


def _shuffle_kernel(pos_ref, emb_ref, hs_ref, mask_ref,
                    out_hs_ref, out_mask_ref,
                    buf, sem_in, sem_out, sem_ctx):
    batch, seq, _ = hs_ref.shape
    n_ctx = emb_ref.shape[1]
    sched = _chunk_schedule(batch, seq)
    n = len(sched)

    def fetch(i):
        b, row, sz = sched[i]
        m = i % _NBUF
        return pltpu.make_async_copy(
            hs_ref.at[pl.ds(b, 1), pl.ds(row, sz), :],
            buf.at[pl.ds(m, 1), pl.ds(0, sz), :], sem_in.at[m])

    def write(i):
        b, row, sz = sched[i]
        m = i % _NBUF
        return pltpu.make_async_copy(
            buf.at[pl.ds(m, 1), pl.ds(0, sz), :],
            out_hs_ref.at[pl.ds(b, 1), pl.ds(n_ctx + row, sz), :],
            sem_out.at[m])

    def ctx_copy(b2):
        p = pos_ref[0]
        return pltpu.make_async_copy(
            emb_ref.at[pl.ds(p, 1)],
            out_hs_ref.at[pl.ds(b2, 1), pl.ds(0, n_ctx), :], sem_ctx)

    for b2 in range(batch):
        ctx_copy(b2).start()

    for i in range(_LOOKAHEAD):
        fetch(i).start()

    for i in range(n):
        j = i + _LOOKAHEAD
        if j < n:
            if j - _NBUF >= 0:
                write(j - _NBUF).wait()
            fetch(j).start()
        fetch(i).wait()
        write(i).start()

    out_mask_ref[:, :n_ctx] = jnp.ones_like(out_mask_ref[:, :n_ctx])
    out_mask_ref[:, n_ctx:] = mask_ref[:, :]

    for i in range(max(0, n - _NBUF), n):
        write(i).wait()
    for b2 in range(batch):
        ctx_copy(b2).wait()


def kernel(hidden_states, attention_mask, position, embeddings):
    B, S, H = hidden_states.shape
    D, T, _ = embeddings.shape
    pos = jnp.asarray(position, dtype=jnp.int32).reshape((1,))

    out_hs, out_mask = pl.pallas_call(
        _shuffle_kernel,
        in_specs=[
            pl.BlockSpec(memory_space=pltpu.SMEM),
            pl.BlockSpec(memory_space=pl.ANY),
            pl.BlockSpec(memory_space=pl.ANY),
            pl.BlockSpec(memory_space=pltpu.VMEM),
        ],
        out_specs=[
            pl.BlockSpec(memory_space=pl.ANY),
            pl.BlockSpec(memory_space=pltpu.VMEM),
        ],
        out_shape=[
            jax.ShapeDtypeStruct((B, T + S, H), hidden_states.dtype),
            jax.ShapeDtypeStruct((B, T + S), attention_mask.dtype),
        ],
        scratch_shapes=[
            pltpu.VMEM((_NBUF, _MID_CHUNK, H), hidden_states.dtype),
            pltpu.SemaphoreType.DMA((_NBUF,)),
            pltpu.SemaphoreType.DMA((_NBUF,)),
            pltpu.SemaphoreType.DMA,
        ],
    )(pos, embeddings, hidden_states, attention_mask)
    return out_hs, out_mask
